# baseline (speedup 1.0000x reference)
import functools

import jax
import jax.numpy as jnp
from jax import lax
from jax.experimental import pallas as pl
from jax.experimental.pallas import tpu as pltpu
from jax.experimental.pallas import tpu_sc as plsc

B = 256
NCHUNK = 7


def _routing(species, n_species, n_pad, nblk):
    n = species.shape[0]
    oh = (species[:, None] == jnp.arange(n_species, dtype=species.dtype)[None, :]
          ).astype(jnp.int32)
    counts = jnp.sum(oh, axis=0)
    pad_counts = ((counts + B - 1) // B) * B
    ends = jnp.cumsum(pad_counts)
    offs = ends - pad_counts
    rstarts = jnp.cumsum(counts) - counts
    sorted_ids = jnp.argsort(species, stable=True).astype(jnp.int32)
    block_start = jnp.arange(nblk, dtype=jnp.int32) * B
    bspec = jnp.sum((block_start[:, None] >= ends[None, :]).astype(jnp.int32),
                    axis=1)
    bspec = jnp.minimum(bspec, n_species - 1).astype(jnp.int32)
    valid = jnp.clip((offs + counts)[bspec] - block_start, 0, B).astype(jnp.int32)
    p = jnp.arange(n_pad, dtype=jnp.int32)
    s_of_p = jnp.repeat(bspec, B)
    local = p - offs[s_of_p]
    validp = local < counts[s_of_p]
    r = rstarts[s_of_p] + local
    r = jnp.where(validp, r, p % n)
    gidx = jnp.take(sorted_ids, jnp.minimum(r, n - 1))
    return gidx, bspec, valid


def _sc_gather(aev, gidx_slice, n_rows, d, dtype=jnp.float32):
    info = plsc.get_sparse_core_info()
    nw = info.num_cores * info.num_subcores
    b_per_w = n_rows // nw
    ch = 16
    nch = b_per_w // ch
    assert b_per_w % ch == 0
    mesh = plsc.VectorSubcoreMesh(core_axis_name="c", subcore_axis_name="s")

    @functools.partial(
        pl.kernel, mesh=mesh,
        out_type=jax.ShapeDtypeStruct((n_rows, d), dtype),
        scratch_types=[
            pltpu.VMEM((b_per_w,), jnp.int32),
            pltpu.VMEM((ch, d), dtype),
            pltpu.VMEM((ch, d), dtype),
            pltpu.VMEM((ch, d), dtype),
            pltpu.SemaphoreType.DMA,
            pltpu.SemaphoreType.DMA,
            pltpu.SemaphoreType.DMA,
            pltpu.SemaphoreType.DMA,
            pltpu.SemaphoreType.DMA,
            pltpu.SemaphoreType.DMA,
        ],
    )
    def gather_k(aev_hbm, gidx_hbm, out_hbm, idx_v, b0, b1, b2,
                 gs0, gs1, gs2, ws0, ws1, ws2):
        bufs = (b0, b1, b2)
        gsems = (gs0, gs1, gs2)
        wsems = (ws0, ws1, ws2)
        wid = lax.axis_index("s") * info.num_cores + lax.axis_index("c")
        base = wid * b_per_w
        pltpu.sync_copy(gidx_hbm.at[pl.ds(base, b_per_w)], idx_v)

        def g_start(j, buf, sem):
            pltpu.async_copy(aev_hbm.at[idx_v.at[pl.ds(j * ch, ch)]], buf, sem)

        def g_wait(buf, sem):
            pltpu.make_async_copy(aev_hbm.at[pl.ds(0, ch)], buf, sem).wait()

        def w_start(j, buf, sem):
            pltpu.async_copy(buf, out_hbm.at[pl.ds(base + j * ch, ch)], sem)

        def w_wait(buf, sem):
            pltpu.make_async_copy(buf, out_hbm.at[pl.ds(base, ch)], sem).wait()

        for j in range(min(3, nch)):
            g_start(j, bufs[j % 3], gsems[j % 3])
        for j in range(nch):
            b = j % 3
            g_wait(bufs[b], gsems[b])
            w_start(j, bufs[b], wsems[b])
            if j + 3 < nch:
                w_wait(bufs[b], wsems[b])
                g_start(j + 3, bufs[b], gsems[b])
        for j in range(max(0, nch - 3), nch):
            w_wait(bufs[j % 3], wsems[j % 3])

    return gather_k(aev, gidx_slice)


def _tc_mlp(sorted_aev, bspec, valid, W1, b1, W2, b2, W3r, b3, nblk):
    kh = W1.shape[1]
    h1 = W1.shape[2]
    h2 = W2.shape[2]

    def body(bs_ref, valid_ref, x_ref, w1_ref, b1_ref, w2_ref,
             b2_ref, w3_ref, b3_ref, out_ref):
        i = pl.program_id(0)
        v = valid_ref[i]
        x = x_ref[...].astype(jnp.bfloat16)
        h = lax.dot_general(x, w1_ref[0], (((1,), (0,)), ((), ())),
                            preferred_element_type=jnp.float32)
        h = h + b1_ref[0]
        h = jnp.where(h > 0, h, jnp.exp(h) - 1.0)
        g = lax.dot_general(h.astype(jnp.bfloat16), w2_ref[0],
                            (((1,), (0,)), ((), ())),
                            preferred_element_type=jnp.float32)
        g = g + b2_ref[0]
        g = jnp.where(g > 0, g, jnp.exp(g) - 1.0)
        rows = lax.broadcasted_iota(jnp.int32, (B, h2), 0)
        maskw = jnp.where(rows < v, 1.0, 0.0)
        g = g * maskw
        colsum = jnp.sum(g, axis=0, keepdims=True)
        cnt = jnp.sum(maskw, axis=0, keepdims=True)
        es = lax.dot_general(colsum, w3_ref[0], (((1,), (1,)), ((), ())),
                             preferred_element_type=jnp.float32)
        es = es + lax.dot_general(cnt, b3_ref[0], (((1,), (1,)), ((), ())),
                                  preferred_element_type=jnp.float32)

        @pl.when(i == 0)
        def _():
            out_ref[...] = jnp.zeros_like(out_ref)

        out_ref[...] += es

    grid_spec = pltpu.PrefetchScalarGridSpec(
        num_scalar_prefetch=2,
        grid=(nblk,),
        in_specs=[
            pl.BlockSpec((B, kh), lambda i, bs, vl: (i, 0)),
            pl.BlockSpec((1, kh, h1), lambda i, bs, vl: (bs[i], 0, 0)),
            pl.BlockSpec((1, 1, h1), lambda i, bs, vl: (bs[i], 0, 0)),
            pl.BlockSpec((1, h1, h2), lambda i, bs, vl: (bs[i], 0, 0)),
            pl.BlockSpec((1, 1, h2), lambda i, bs, vl: (bs[i], 0, 0)),
            pl.BlockSpec((1, 1, h2), lambda i, bs, vl: (bs[i], 0, 0)),
            pl.BlockSpec((1, 1, h2), lambda i, bs, vl: (bs[i], 0, 0)),
        ],
        out_specs=pl.BlockSpec((1, 1), lambda i, bs, vl: (0, 0)),
    )
    return pl.pallas_call(
        body,
        grid_spec=grid_spec,
        out_shape=jax.ShapeDtypeStruct((1, 1), jnp.float32),
        compiler_params=pltpu.CompilerParams(
            dimension_semantics=("arbitrary",)),
    )(bspec, valid, sorted_aev, W1, b1, W2, b2, W3r, b3)


def kernel(aev, W1, b1, W2, b2, W3, b3, species):
    n, d = aev.shape
    n_species = W1.shape[0]
    nblk = n // B + n_species
    if nblk == 132:
        chunks = [6, 24, 30, 30, 24, 12, 6]
    else:
        nblk += (-nblk) % NCHUNK
        chunks = [nblk // NCHUNK] * NCHUNK
    n_pad = nblk * B
    gidx, bspec, valid = _routing(species, n_species, n_pad, nblk)
    W1b = W1.astype(jnp.bfloat16)
    W2b = W2.astype(jnp.bfloat16)
    b1r = b1.reshape(n_species, 1, -1)
    b2r = b2.reshape(n_species, 1, -1)
    W3r = W3.reshape(n_species, 1, -1)
    h2 = W2.shape[2]
    b3r = jnp.broadcast_to(b3.reshape(n_species, 1, 1) / h2,
                           (n_species, 1, h2))
    total = None
    blk0 = 0
    for nb in chunks:
        gslice = lax.slice_in_dim(gidx, blk0 * B, (blk0 + nb) * B)
        sorted_c = _sc_gather(aev, gslice, nb * B, d)
        bs_c = lax.slice_in_dim(bspec, blk0, blk0 + nb)
        vl_c = lax.slice_in_dim(valid, blk0, blk0 + nb)
        out_c = _tc_mlp(sorted_c, bs_c, vl_c, W1b, b1r, W2b, b2r, W3r,
                        b3r, nb)
        total = out_c if total is None else total + out_c
        blk0 += nb
    return total.reshape(1)


# speedup vs baseline: 1.0367x; 1.0367x over previous
import functools

import jax
import jax.numpy as jnp
from jax import lax
from jax.experimental import pallas as pl
from jax.experimental.pallas import tpu as pltpu
from jax.experimental.pallas import tpu_sc as plsc

B = 256
NCHUNK = 4


def _routing(species, n_species, n_pad, nblk):
    n = species.shape[0]
    oh = (species[:, None] == jnp.arange(n_species, dtype=species.dtype)[None, :]
          ).astype(jnp.int32)
    counts = jnp.sum(oh, axis=0)
    pad_counts = ((counts + B - 1) // B) * B
    ends = jnp.cumsum(pad_counts)
    offs = ends - pad_counts
    rstarts = jnp.cumsum(counts) - counts
    sorted_ids = jnp.argsort(species, stable=True).astype(jnp.int32)
    block_start = jnp.arange(nblk, dtype=jnp.int32) * B
    bspec = jnp.sum((block_start[:, None] >= ends[None, :]).astype(jnp.int32),
                    axis=1)
    bspec = jnp.minimum(bspec, n_species - 1).astype(jnp.int32)
    valid = jnp.clip((offs + counts)[bspec] - block_start, 0, B).astype(jnp.int32)
    p = jnp.arange(n_pad, dtype=jnp.int32)
    s_of_p = jnp.repeat(bspec, B)
    local = p - offs[s_of_p]
    validp = local < counts[s_of_p]
    r = rstarts[s_of_p] + local
    r = jnp.where(validp, r, p % n)
    gidx = jnp.take(sorted_ids, jnp.minimum(r, n - 1))
    return gidx, bspec, valid


def _sc_gather(aev, gidx_slice, n_rows, d, dtype=jnp.float32):
    info = plsc.get_sparse_core_info()
    nw = info.num_cores * info.num_subcores
    b_per_w = n_rows // nw
    ch = 24
    nch = b_per_w // ch
    assert b_per_w % ch == 0
    mesh = plsc.VectorSubcoreMesh(core_axis_name="c", subcore_axis_name="s")

    @functools.partial(
        pl.kernel, mesh=mesh,
        out_type=jax.ShapeDtypeStruct((n_rows, d), dtype),
        scratch_types=[
            pltpu.VMEM((b_per_w,), jnp.int32),
            pltpu.VMEM((ch, d), dtype),
            pltpu.VMEM((ch, d), dtype),
            pltpu.VMEM((ch, d), dtype),
            pltpu.SemaphoreType.DMA,
            pltpu.SemaphoreType.DMA,
            pltpu.SemaphoreType.DMA,
            pltpu.SemaphoreType.DMA,
            pltpu.SemaphoreType.DMA,
            pltpu.SemaphoreType.DMA,
        ],
    )
    def gather_k(aev_hbm, gidx_hbm, out_hbm, idx_v, b0, b1, b2,
                 gs0, gs1, gs2, ws0, ws1, ws2):
        bufs = (b0, b1, b2)
        gsems = (gs0, gs1, gs2)
        wsems = (ws0, ws1, ws2)
        wid = lax.axis_index("s") * info.num_cores + lax.axis_index("c")
        base = wid * b_per_w
        pltpu.sync_copy(gidx_hbm.at[pl.ds(base, b_per_w)], idx_v)

        def g_start(j, buf, sem):
            pltpu.async_copy(aev_hbm.at[idx_v.at[pl.ds(j * ch, ch)]], buf, sem)

        def g_wait(buf, sem):
            pltpu.make_async_copy(aev_hbm.at[pl.ds(0, ch)], buf, sem).wait()

        def w_start(j, buf, sem):
            pltpu.async_copy(buf, out_hbm.at[pl.ds(base + j * ch, ch)], sem)

        def w_wait(buf, sem):
            pltpu.make_async_copy(buf, out_hbm.at[pl.ds(base, ch)], sem).wait()

        for j in range(min(3, nch)):
            g_start(j, bufs[j % 3], gsems[j % 3])
        for j in range(nch):
            b = j % 3
            g_wait(bufs[b], gsems[b])
            w_start(j, bufs[b], wsems[b])
            if j + 3 < nch:
                w_wait(bufs[b], wsems[b])
                g_start(j + 3, bufs[b], gsems[b])
        for j in range(max(0, nch - 3), nch):
            w_wait(bufs[j % 3], wsems[j % 3])

    return gather_k(aev, gidx_slice)


def _tc_mlp(sorted_aev, bspec, valid, W1, b1, W2, b2, W3r, b3, nblk):
    kh = W1.shape[1]
    h1 = W1.shape[2]
    h2 = W2.shape[2]

    def body(bs_ref, valid_ref, x_ref, w1_ref, b1_ref, w2_ref,
             b2_ref, w3_ref, b3_ref, out_ref):
        i = pl.program_id(0)
        v = valid_ref[i]
        x = x_ref[...]
        h = lax.dot_general(x, w1_ref[0], (((1,), (0,)), ((), ())),
                            preferred_element_type=jnp.float32)
        h = h + b1_ref[0]
        h = jnp.where(h > 0, h, jnp.exp(h) - 1.0)
        g = lax.dot_general(h, w2_ref[0], (((1,), (0,)), ((), ())),
                            preferred_element_type=jnp.float32)
        g = g + b2_ref[0]
        g = jnp.where(g > 0, g, jnp.exp(g) - 1.0)
        rows = lax.broadcasted_iota(jnp.int32, (B, h2), 0)
        maskw = jnp.where(rows < v, 1.0, 0.0)
        g = g * maskw
        colsum = jnp.sum(g, axis=0, keepdims=True)
        cnt = jnp.sum(maskw, axis=0, keepdims=True)
        es = lax.dot_general(colsum, w3_ref[0], (((1,), (1,)), ((), ())),
                             preferred_element_type=jnp.float32)
        es = es + lax.dot_general(cnt, b3_ref[0], (((1,), (1,)), ((), ())),
                                  preferred_element_type=jnp.float32)

        @pl.when(i == 0)
        def _():
            out_ref[...] = jnp.zeros_like(out_ref)

        out_ref[...] += es

    grid_spec = pltpu.PrefetchScalarGridSpec(
        num_scalar_prefetch=2,
        grid=(nblk,),
        in_specs=[
            pl.BlockSpec((B, kh), lambda i, bs, vl: (i, 0)),
            pl.BlockSpec((1, kh, h1), lambda i, bs, vl: (bs[i], 0, 0)),
            pl.BlockSpec((1, 1, h1), lambda i, bs, vl: (bs[i], 0, 0)),
            pl.BlockSpec((1, h1, h2), lambda i, bs, vl: (bs[i], 0, 0)),
            pl.BlockSpec((1, 1, h2), lambda i, bs, vl: (bs[i], 0, 0)),
            pl.BlockSpec((1, 1, h2), lambda i, bs, vl: (bs[i], 0, 0)),
            pl.BlockSpec((1, 1, h2), lambda i, bs, vl: (bs[i], 0, 0)),
        ],
        out_specs=pl.BlockSpec((1, 1), lambda i, bs, vl: (0, 0)),
    )
    return pl.pallas_call(
        body,
        grid_spec=grid_spec,
        out_shape=jax.ShapeDtypeStruct((1, 1), jnp.float32),
        compiler_params=pltpu.CompilerParams(
            dimension_semantics=("arbitrary",)),
    )(bspec, valid, sorted_aev, W1, b1, W2, b2, W3r, b3)


def kernel(aev, W1, b1, W2, b2, W3, b3, species):
    n, d = aev.shape
    n_species = W1.shape[0]
    nblk = n // B + n_species
    if nblk == 132:
        chunks = [33, 33, 33, 33]
    else:
        nblk += (-nblk) % NCHUNK
        chunks = [nblk // NCHUNK] * NCHUNK
    n_pad = nblk * B
    gidx, bspec, valid = _routing(species, n_species, n_pad, nblk)
    b1r = b1.reshape(n_species, 1, -1)
    b2r = b2.reshape(n_species, 1, -1)
    W3r = W3.reshape(n_species, 1, -1)
    h2 = W2.shape[2]
    b3r = jnp.broadcast_to(b3.reshape(n_species, 1, 1) / h2,
                           (n_species, 1, h2))
    total = None
    blk0 = 0
    for nb in chunks:
        gslice = lax.slice_in_dim(gidx, blk0 * B, (blk0 + nb) * B)
        sorted_c = _sc_gather(aev, gslice, nb * B, d)
        bs_c = lax.slice_in_dim(bspec, blk0, blk0 + nb)
        vl_c = lax.slice_in_dim(valid, blk0, blk0 + nb)
        out_c = _tc_mlp(sorted_c, bs_c, vl_c, W1, b1r, W2, b2r, W3r,
                        b3r, nb)
        total = out_c if total is None else total + out_c
        blk0 += nb
    return total.reshape(1)
